# baseline (speedup 1.0000x reference)
import functools

import jax
import jax.numpy as jnp
from jax import lax
from jax.experimental import pallas as pl
from jax.experimental.pallas import tpu as pltpu
from jax.experimental.pallas import tpu_sc as plsc

_B = 16384
_D = 16
_NC = 2
_NS = 16
_NW = _NC * _NS
_BPW = _B // _NW
_CHUNK = 128
_NCH = _BPW // _CHUNK
_REG = 0.0001


@functools.partial(
    pl.kernel,
    out_type=[
        jax.ShapeDtypeStruct((_B,), jnp.float32),
        jax.ShapeDtypeStruct((_NW, 3, _D), jnp.float32),
    ],
    mesh=plsc.VectorSubcoreMesh(core_axis_name="c", subcore_axis_name="s"),
    compiler_params=pltpu.CompilerParams(
        needs_layout_passes=False, use_tc_tiling_on_sc=False
    ),
    scratch_types=[
        pltpu.VMEM((_NCH, _CHUNK), jnp.int32),
        pltpu.VMEM((_NCH, _CHUNK), jnp.int32),
        pltpu.VMEM((_BPW, _D), jnp.float32),
        pltpu.VMEM((_BPW, _D), jnp.float32),
        pltpu.VMEM((_BPW, _D), jnp.float32),
        pltpu.VMEM((_BPW,), jnp.float32),
        pltpu.VMEM((3, _D), jnp.float32),
        pltpu.VMEM((3, _D), jnp.float32),
        pltpu.SemaphoreType.DMA,
    ],
)
def _sc_body(ps_hbm, rs_hbm, p_hbm, q_hbm, r_hbm, fc_hbm,
             out_hbm, part_hbm,
             idxp_v, idxr_v, pv, qv, rv, outv, partv, fcv, sem):
    wid = lax.axis_index("s") * _NC + lax.axis_index("c")
    base = wid * _BPW

    pltpu.sync_copy(ps_hbm.at[pl.ds(wid * _NCH, _NCH)], idxp_v)
    pltpu.sync_copy(rs_hbm.at[pl.ds(wid * _NCH, _NCH)], idxr_v)
    pltpu.sync_copy(fc_hbm, fcv)

    copies = []
    for j in range(_NCH):
        dst = pl.ds(j * _CHUNK, _CHUNK)
        copies.append(pltpu.async_copy(p_hbm.at[idxp_v.at[j]], pv.at[dst], sem))
        copies.append(pltpu.async_copy(q_hbm.at[idxp_v.at[j]], qv.at[dst], sem))
        copies.append(pltpu.async_copy(r_hbm.at[idxr_v.at[j]], rv.at[dst], sem))
    for c in copies:
        c.wait()

    fpv, fqv, frv = fcv[0], fcv[1], fcv[2]
    fps = [fpv[d] for d in range(_D)]
    fqs = [fqv[d] for d in range(_D)]
    frs = [frv[d] for d in range(_D)]
    lane = lax.iota(jnp.int32, _D)
    zero = jnp.zeros((_D,), jnp.float32)

    def block(b, carry):
        sp, sq, sr = carry
        i0 = b * _D
        rows = i0 + lane
        acc = zero
        for d in range(_D):
            cols = jnp.full((_D,), d, jnp.int32)
            colp = plsc.load_gather(pv, [rows, cols])
            colq = plsc.load_gather(qv, [rows, cols])
            colr = plsc.load_gather(rv, [rows, cols])
            acc = acc + colp * fps[d] + colq * fqs[d] + colr * frs[d]
            sp = sp + colp * colp
            sq = sq + colq * colq
            sr = sr + colr * colr
        outv[pl.ds(i0, _D)] = acc
        return (sp, sq, sr)

    sp, sq, sr = lax.fori_loop(0, _BPW // _D, block, (zero, zero, zero))
    partv[0] = sp
    partv[1] = sq
    partv[2] = sr

    pltpu.sync_copy(outv, out_hbm.at[pl.ds(base, _BPW)])
    pltpu.sync_copy(partv, part_hbm.at[wid])


def kernel(ps, qs, rs, P_table, Q_table, R_table, wp, wq, wr, fc_w):
    del qs
    fc = fc_w[0].astype(jnp.float32)
    fc_c = fc / jnp.maximum(jnp.sqrt(jnp.sum(fc * fc)), 1.0)

    def _cw(w):
        s = w[0, 0]
        return s / jnp.maximum(jnp.abs(s), 1.0)

    fcs = jnp.stack([fc_c * _cw(wp), fc_c * _cw(wq), fc_c * _cw(wr)])

    ps2 = ps.astype(jnp.int32).reshape(_NW * _NCH, _CHUNK)
    rs2 = rs.astype(jnp.int32).reshape(_NW * _NCH, _CHUNK)

    out, parts = _sc_body(ps2, rs2, P_table, Q_table, R_table, fcs)

    inferences = out.reshape(_B, 1)
    sums = parts.sum(axis=(0, 2))
    regs = _REG * (jnp.sqrt(sums[0]) + jnp.sqrt(sums[1]) + jnp.sqrt(sums[2]))
    return (inferences, regs)


# speedup vs baseline: 8.2883x; 8.2883x over previous
import functools

import jax
import jax.numpy as jnp
from jax import lax
from jax.experimental import pallas as pl
from jax.experimental.pallas import tpu as pltpu
from jax.experimental.pallas import tpu_sc as plsc

_V = 1000000
_B = 16384
_D = 16
_NC = 2
_NS = 16
_NW = _NC * _NS
_BPW = _B // _NW
_CHUNK = 128
_NCH = _BPW // _CHUNK
_VC = 16384
_GA = -(-_V // _VC)
_REG = 0.0001


def _tc_body(pt, qt, rt, w, spq, sr, np_, nq_, nr):
    p = pt[...]
    q = qt[...]
    r = rt[...]
    fcp = w[0, :].reshape(_D, 1)
    fcq = w[1, :].reshape(_D, 1)
    fcr = w[2, :].reshape(_D, 1)
    spq[...] = jnp.sum(p * fcp + q * fcq, axis=0)
    sr[...] = jnp.sum(r * fcr, axis=0)
    np_[...] = jnp.sum(p * p, axis=0)
    nq_[...] = jnp.sum(q * q, axis=0)
    nr[...] = jnp.sum(r * r, axis=0)


@functools.partial(
    pl.kernel,
    out_type=[
        jax.ShapeDtypeStruct((_B,), jnp.float32),
        jax.ShapeDtypeStruct((_NW, 3, _D), jnp.float32),
    ],
    mesh=plsc.VectorSubcoreMesh(core_axis_name="c", subcore_axis_name="s"),
    scratch_types=[
        pltpu.VMEM((_NCH, _CHUNK), jnp.int32),
        pltpu.VMEM((_NCH, _CHUNK), jnp.int32),
        pltpu.VMEM((_BPW,), jnp.float32),
        pltpu.VMEM((_BPW,), jnp.float32),
        pltpu.VMEM((_BPW,), jnp.float32),
        pltpu.VMEM((_BPW,), jnp.float32),
        pltpu.VMEM((_BPW,), jnp.float32),
        pltpu.VMEM((_BPW,), jnp.float32),
        pltpu.VMEM((3, _D), jnp.float32),
        pltpu.SemaphoreType.DMA,
    ],
)
def _sc_body(ps_hbm, rs_hbm, spq_hbm, sr_hbm, np_hbm, nq_hbm, nr_hbm,
             out_hbm, part_hbm,
             idxp_v, idxr_v, spq_v, sr_v, np_v, nq_v, nr_v, outv, partv, sem):
    wid = lax.axis_index("s") * _NC + lax.axis_index("c")
    base = wid * _BPW

    pltpu.sync_copy(ps_hbm.at[pl.ds(wid * _NCH, _NCH)], idxp_v)
    pltpu.sync_copy(rs_hbm.at[pl.ds(wid * _NCH, _NCH)], idxr_v)

    copies = []
    for j in range(_NCH):
        dst = pl.ds(j * _CHUNK, _CHUNK)
        ip = idxp_v.at[j]
        ir = idxr_v.at[j]
        copies.append(pltpu.async_copy(spq_hbm.at[ip], spq_v.at[dst], sem))
        copies.append(pltpu.async_copy(np_hbm.at[ip], np_v.at[dst], sem))
        copies.append(pltpu.async_copy(nq_hbm.at[ip], nq_v.at[dst], sem))
        copies.append(pltpu.async_copy(sr_hbm.at[ir], sr_v.at[dst], sem))
        copies.append(pltpu.async_copy(nr_hbm.at[ir], nr_v.at[dst], sem))
    for c in copies:
        c.wait()

    zero = jnp.zeros((_D,), jnp.float32)

    def blk(i, carry):
        ap, aq, ar = carry
        ds = pl.ds(i * _D, _D)
        outv[ds] = spq_v[ds] + sr_v[ds]
        return (ap + np_v[ds], aq + nq_v[ds], ar + nr_v[ds])

    ap, aq, ar = lax.fori_loop(0, _BPW // _D, blk, (zero, zero, zero))
    partv[0] = ap
    partv[1] = aq
    partv[2] = ar

    pltpu.sync_copy(outv, out_hbm.at[pl.ds(base, _BPW)])
    pltpu.sync_copy(partv, part_hbm.at[wid])


def kernel(ps, qs, rs, P_table, Q_table, R_table, wp, wq, wr, fc_w):
    del qs
    fc = fc_w[0].astype(jnp.float32)
    fc_c = fc / jnp.maximum(jnp.sqrt(jnp.sum(fc * fc)), 1.0)

    def _cw(w):
        s = w[0, 0]
        return s / jnp.maximum(jnp.abs(s), 1.0)

    fcs = jnp.stack([fc_c * _cw(wp), fc_c * _cw(wq), fc_c * _cw(wr)])

    spq, sr, np_, nq_, nr = pl.pallas_call(
        _tc_body,
        grid=(_GA,),
        in_specs=[
            pl.BlockSpec((_D, _VC), lambda c: (0, c)),
            pl.BlockSpec((_D, _VC), lambda c: (0, c)),
            pl.BlockSpec((_D, _VC), lambda c: (0, c)),
            pl.BlockSpec((3, _D), lambda c: (0, 0)),
        ],
        out_specs=[pl.BlockSpec((_VC,), lambda c: (c,))] * 5,
        out_shape=[jax.ShapeDtypeStruct((_V,), jnp.float32)] * 5,
    )(P_table.T, Q_table.T, R_table.T, fcs)

    ps2 = ps.astype(jnp.int32).reshape(_NW * _NCH, _CHUNK)
    rs2 = rs.astype(jnp.int32).reshape(_NW * _NCH, _CHUNK)

    out, parts = _sc_body(ps2, rs2, spq, sr, np_, nq_, nr)

    inferences = out.reshape(_B, 1)
    sums = parts.sum(axis=(0, 2))
    regs = _REG * (jnp.sqrt(sums[0]) + jnp.sqrt(sums[1]) + jnp.sqrt(sums[2]))
    return (inferences, regs)


# speedup vs baseline: 9.3207x; 1.1246x over previous
import functools

import jax
import jax.numpy as jnp
from jax import lax
from jax.experimental import pallas as pl
from jax.experimental.pallas import tpu as pltpu
from jax.experimental.pallas import tpu_sc as plsc

_V = 1000000
_B = 16384
_D = 16
_NC = 2
_NS = 16
_NW = _NC * _NS
_BPW = _B // _NW
_CHUNK = 128
_NCH = _BPW // _CHUNK
_VC = 16384
_GA = -(-_V // _VC)
_REG = 0.0001


def _tc_body(pt, qt, rt, w, spq, sr, np_, nq_, nr):
    p = pt[...]
    q = qt[...]
    r = rt[...]
    ww = w[...]
    dn = (((1,), (0,)), ((), ()))
    f32 = jnp.float32
    mp = lax.dot_general(ww, p, dn, preferred_element_type=f32)
    mq = lax.dot_general(ww, q, dn, preferred_element_type=f32)
    mr = lax.dot_general(ww, r, dn, preferred_element_type=f32)
    m2p = lax.dot_general(ww, p * p, dn, preferred_element_type=f32)
    m2q = lax.dot_general(ww, q * q, dn, preferred_element_type=f32)
    m2r = lax.dot_general(ww, r * r, dn, preferred_element_type=f32)
    spq[...] = mp[0] + mq[1]
    sr[...] = mr[2]
    np_[...] = m2p[3]
    nq_[...] = m2q[3]
    nr[...] = m2r[3]


@functools.partial(
    pl.kernel,
    out_type=[
        jax.ShapeDtypeStruct((_B,), jnp.float32),
        jax.ShapeDtypeStruct((_NW, 3, _D), jnp.float32),
    ],
    mesh=plsc.VectorSubcoreMesh(core_axis_name="c", subcore_axis_name="s"),
    scratch_types=[
        pltpu.VMEM((_NCH, _CHUNK), jnp.int32),
        pltpu.VMEM((_NCH, _CHUNK), jnp.int32),
        pltpu.VMEM((_BPW,), jnp.float32),
        pltpu.VMEM((_BPW,), jnp.float32),
        pltpu.VMEM((_BPW,), jnp.float32),
        pltpu.VMEM((_BPW,), jnp.float32),
        pltpu.VMEM((_BPW,), jnp.float32),
        pltpu.VMEM((_BPW,), jnp.float32),
        pltpu.VMEM((3, _D), jnp.float32),
        pltpu.SemaphoreType.DMA,
    ],
)
def _sc_body(ps_hbm, rs_hbm, spq_hbm, sr_hbm, np_hbm, nq_hbm, nr_hbm,
             out_hbm, part_hbm,
             idxp_v, idxr_v, spq_v, sr_v, np_v, nq_v, nr_v, outv, partv, sem):
    wid = lax.axis_index("s") * _NC + lax.axis_index("c")
    base = wid * _BPW

    pltpu.sync_copy(ps_hbm.at[pl.ds(wid * _NCH, _NCH)], idxp_v)
    pltpu.sync_copy(rs_hbm.at[pl.ds(wid * _NCH, _NCH)], idxr_v)

    copies = []
    for j in range(_NCH):
        dst = pl.ds(j * _CHUNK, _CHUNK)
        ip = idxp_v.at[j]
        ir = idxr_v.at[j]
        copies.append(pltpu.async_copy(spq_hbm.at[ip], spq_v.at[dst], sem))
        copies.append(pltpu.async_copy(np_hbm.at[ip], np_v.at[dst], sem))
        copies.append(pltpu.async_copy(nq_hbm.at[ip], nq_v.at[dst], sem))
        copies.append(pltpu.async_copy(sr_hbm.at[ir], sr_v.at[dst], sem))
        copies.append(pltpu.async_copy(nr_hbm.at[ir], nr_v.at[dst], sem))
    for c in copies:
        c.wait()

    zero = jnp.zeros((_D,), jnp.float32)

    def blk(i, carry):
        ap, aq, ar = carry
        ds = pl.ds(i * _D, _D)
        outv[ds] = spq_v[ds] + sr_v[ds]
        return (ap + np_v[ds], aq + nq_v[ds], ar + nr_v[ds])

    ap, aq, ar = lax.fori_loop(0, _BPW // _D, blk, (zero, zero, zero))
    partv[0] = ap
    partv[1] = aq
    partv[2] = ar

    pltpu.sync_copy(outv, out_hbm.at[pl.ds(base, _BPW)])
    pltpu.sync_copy(partv, part_hbm.at[wid])


def kernel(ps, qs, rs, P_table, Q_table, R_table, wp, wq, wr, fc_w):
    del qs
    fc = fc_w[0].astype(jnp.float32)
    fc_c = fc / jnp.maximum(jnp.sqrt(jnp.sum(fc * fc)), 1.0)

    def _cw(w):
        s = w[0, 0]
        return s / jnp.maximum(jnp.abs(s), 1.0)

    fcs = jnp.concatenate([
        jnp.stack([fc_c * _cw(wp), fc_c * _cw(wq), fc_c * _cw(wr),
                   jnp.ones((_D,), jnp.float32)]),
        jnp.zeros((4, _D), jnp.float32),
    ])

    spq, sr, np_, nq_, nr = pl.pallas_call(
        _tc_body,
        grid=(_GA,),
        in_specs=[
            pl.BlockSpec((_D, _VC), lambda c: (0, c)),
            pl.BlockSpec((_D, _VC), lambda c: (0, c)),
            pl.BlockSpec((_D, _VC), lambda c: (0, c)),
            pl.BlockSpec((8, _D), lambda c: (0, 0)),
        ],
        out_specs=[pl.BlockSpec((_VC,), lambda c: (c,))] * 5,
        out_shape=[jax.ShapeDtypeStruct((_V,), jnp.float32)] * 5,
    )(P_table.T, Q_table.T, R_table.T, fcs)

    ps2 = ps.astype(jnp.int32).reshape(_NW * _NCH, _CHUNK)
    rs2 = rs.astype(jnp.int32).reshape(_NW * _NCH, _CHUNK)

    out, parts = _sc_body(ps2, rs2, spq, sr, np_, nq_, nr)

    inferences = out.reshape(_B, 1)
    sums = parts.sum(axis=(0, 2))
    regs = _REG * (jnp.sqrt(sums[0]) + jnp.sqrt(sums[1]) + jnp.sqrt(sums[2]))
    return (inferences, regs)


# speedup vs baseline: 11.0581x; 1.1864x over previous
import functools

import jax
import jax.numpy as jnp
from jax import lax
from jax.experimental import pallas as pl
from jax.experimental.pallas import tpu as pltpu
from jax.experimental.pallas import tpu_sc as plsc

_V = 1000000
_B = 16384
_D = 16
_NC = 2
_NS = 16
_NW = _NC * _NS
_BPW = _B // _NW
_CHUNK = 128
_NCH = _BPW // _CHUNK
_VC = 32768
_GA = -(-_V // _VC)
_REG = 0.0001


def _tc_body(pt, qt, rt, w, spq, sr, np_, nq_, nr):
    p = pt[...]
    q = qt[...]
    r = rt[...]
    ww = w[...]
    dn = (((1,), (0,)), ((), ()))
    f32 = jnp.float32
    mp = lax.dot_general(ww, p, dn, preferred_element_type=f32)
    mq = lax.dot_general(ww, q, dn, preferred_element_type=f32)
    mr = lax.dot_general(ww, r, dn, preferred_element_type=f32)
    m2p = lax.dot_general(ww, p * p, dn, preferred_element_type=f32)
    m2q = lax.dot_general(ww, q * q, dn, preferred_element_type=f32)
    m2r = lax.dot_general(ww, r * r, dn, preferred_element_type=f32)
    spq[...] = mp[0] + mq[1]
    sr[...] = mr[2]
    np_[...] = m2p[3]
    nq_[...] = m2q[3]
    nr[...] = m2r[3]


@functools.partial(
    pl.kernel,
    out_type=[
        jax.ShapeDtypeStruct((_B,), jnp.float32),
        jax.ShapeDtypeStruct((_NW, 3, _D), jnp.float32),
    ],
    mesh=plsc.VectorSubcoreMesh(core_axis_name="c", subcore_axis_name="s"),
    scratch_types=[
        pltpu.VMEM((_NCH, _CHUNK), jnp.int32),
        pltpu.VMEM((_NCH, _CHUNK), jnp.int32),
        pltpu.VMEM((_BPW,), jnp.float32),
        pltpu.VMEM((_BPW,), jnp.float32),
        pltpu.VMEM((_BPW,), jnp.float32),
        pltpu.VMEM((_BPW,), jnp.float32),
        pltpu.VMEM((_BPW,), jnp.float32),
        pltpu.VMEM((_BPW,), jnp.float32),
        pltpu.VMEM((3, _D), jnp.float32),
        pltpu.SemaphoreType.DMA,
    ],
)
def _sc_body(ps_hbm, rs_hbm, spq_hbm, sr_hbm, np_hbm, nq_hbm, nr_hbm,
             out_hbm, part_hbm,
             idxp_v, idxr_v, spq_v, sr_v, np_v, nq_v, nr_v, outv, partv, sem):
    wid = lax.axis_index("s") * _NC + lax.axis_index("c")
    base = wid * _BPW

    pltpu.sync_copy(ps_hbm.at[pl.ds(wid * _NCH, _NCH)], idxp_v)
    pltpu.sync_copy(rs_hbm.at[pl.ds(wid * _NCH, _NCH)], idxr_v)

    copies = []
    for j in range(_NCH):
        dst = pl.ds(j * _CHUNK, _CHUNK)
        ip = idxp_v.at[j]
        ir = idxr_v.at[j]
        copies.append(pltpu.async_copy(spq_hbm.at[ip], spq_v.at[dst], sem))
        copies.append(pltpu.async_copy(np_hbm.at[ip], np_v.at[dst], sem))
        copies.append(pltpu.async_copy(nq_hbm.at[ip], nq_v.at[dst], sem))
        copies.append(pltpu.async_copy(sr_hbm.at[ir], sr_v.at[dst], sem))
        copies.append(pltpu.async_copy(nr_hbm.at[ir], nr_v.at[dst], sem))
    for c in copies:
        c.wait()

    zero = jnp.zeros((_D,), jnp.float32)

    def blk(i, carry):
        ap, aq, ar = carry
        ds = pl.ds(i * _D, _D)
        outv[ds] = spq_v[ds] + sr_v[ds]
        return (ap + np_v[ds], aq + nq_v[ds], ar + nr_v[ds])

    ap, aq, ar = lax.fori_loop(0, _BPW // _D, blk, (zero, zero, zero))
    partv[0] = ap
    partv[1] = aq
    partv[2] = ar

    pltpu.sync_copy(outv, out_hbm.at[pl.ds(base, _BPW)])
    pltpu.sync_copy(partv, part_hbm.at[wid])


def kernel(ps, qs, rs, P_table, Q_table, R_table, wp, wq, wr, fc_w):
    del qs
    fc = fc_w[0].astype(jnp.float32)
    fc_c = fc / jnp.maximum(jnp.sqrt(jnp.sum(fc * fc)), 1.0)

    def _cw(w):
        s = w[0, 0]
        return s / jnp.maximum(jnp.abs(s), 1.0)

    fcs = jnp.concatenate([
        jnp.stack([fc_c * _cw(wp), fc_c * _cw(wq), fc_c * _cw(wr),
                   jnp.ones((_D,), jnp.float32)]),
        jnp.zeros((4, _D), jnp.float32),
    ])

    spq, sr, np_, nq_, nr = pl.pallas_call(
        _tc_body,
        grid=(_GA,),
        in_specs=[
            pl.BlockSpec((_D, _VC), lambda c: (0, c)),
            pl.BlockSpec((_D, _VC), lambda c: (0, c)),
            pl.BlockSpec((_D, _VC), lambda c: (0, c)),
            pl.BlockSpec((8, _D), lambda c: (0, 0)),
        ],
        out_specs=[pl.BlockSpec((_VC,), lambda c: (c,))] * 5,
        out_shape=[jax.ShapeDtypeStruct((_V,), jnp.float32)] * 5,
    )(P_table.T, Q_table.T, R_table.T, fcs)

    ps2 = ps.astype(jnp.int32).reshape(_NW * _NCH, _CHUNK)
    rs2 = rs.astype(jnp.int32).reshape(_NW * _NCH, _CHUNK)

    out, parts = _sc_body(ps2, rs2, spq, sr, np_, nq_, nr)

    inferences = out.reshape(_B, 1)
    sums = parts.sum(axis=(0, 2))
    regs = _REG * (jnp.sqrt(sums[0]) + jnp.sqrt(sums[1]) + jnp.sqrt(sums[2]))
    return (inferences, regs)


# speedup vs baseline: 11.6713x; 1.0555x over previous
import functools

import jax
import jax.numpy as jnp
from jax import lax
from jax.experimental import pallas as pl
from jax.experimental.pallas import tpu as pltpu
from jax.experimental.pallas import tpu_sc as plsc

_V = 1000000
_B = 16384
_D = 16
_NC = 2
_NS = 16
_NW = _NC * _NS
_BPW = _B // _NW
_CHUNK = 128
_NCH = _BPW // _CHUNK
_VC = 65536
_GA = -(-_V // _VC)
_REG = 0.0001


def _tc_body(pt, qt, rt, w, spq, sr, np_, nq_, nr):
    p = pt[...]
    q = qt[...]
    r = rt[...]
    ww = w[...]
    dn = (((1,), (0,)), ((), ()))
    f32 = jnp.float32
    mp = lax.dot_general(ww, p, dn, preferred_element_type=f32)
    mq = lax.dot_general(ww, q, dn, preferred_element_type=f32)
    mr = lax.dot_general(ww, r, dn, preferred_element_type=f32)
    m2p = lax.dot_general(ww, p * p, dn, preferred_element_type=f32)
    m2q = lax.dot_general(ww, q * q, dn, preferred_element_type=f32)
    m2r = lax.dot_general(ww, r * r, dn, preferred_element_type=f32)
    spq[...] = mp[0] + mq[1]
    sr[...] = mr[2]
    np_[...] = m2p[3]
    nq_[...] = m2q[3]
    nr[...] = m2r[3]


@functools.partial(
    pl.kernel,
    out_type=[
        jax.ShapeDtypeStruct((_B,), jnp.float32),
        jax.ShapeDtypeStruct((_NW, 3, _D), jnp.float32),
    ],
    mesh=plsc.VectorSubcoreMesh(core_axis_name="c", subcore_axis_name="s"),
    scratch_types=[
        pltpu.VMEM((_NCH, _CHUNK), jnp.int32),
        pltpu.VMEM((_NCH, _CHUNK), jnp.int32),
        pltpu.VMEM((_BPW,), jnp.float32),
        pltpu.VMEM((_BPW,), jnp.float32),
        pltpu.VMEM((_BPW,), jnp.float32),
        pltpu.VMEM((_BPW,), jnp.float32),
        pltpu.VMEM((_BPW,), jnp.float32),
        pltpu.VMEM((_BPW,), jnp.float32),
        pltpu.VMEM((3, _D), jnp.float32),
        pltpu.SemaphoreType.DMA,
    ],
)
def _sc_body(ps_hbm, rs_hbm, spq_hbm, sr_hbm, np_hbm, nq_hbm, nr_hbm,
             out_hbm, part_hbm,
             idxp_v, idxr_v, spq_v, sr_v, np_v, nq_v, nr_v, outv, partv, sem):
    wid = lax.axis_index("s") * _NC + lax.axis_index("c")
    base = wid * _BPW

    pltpu.sync_copy(ps_hbm.at[pl.ds(wid * _NCH, _NCH)], idxp_v)
    pltpu.sync_copy(rs_hbm.at[pl.ds(wid * _NCH, _NCH)], idxr_v)

    copies = []
    for j in range(_NCH):
        dst = pl.ds(j * _CHUNK, _CHUNK)
        ip = idxp_v.at[j]
        ir = idxr_v.at[j]
        copies.append(pltpu.async_copy(spq_hbm.at[ip], spq_v.at[dst], sem))
        copies.append(pltpu.async_copy(np_hbm.at[ip], np_v.at[dst], sem))
        copies.append(pltpu.async_copy(nq_hbm.at[ip], nq_v.at[dst], sem))
        copies.append(pltpu.async_copy(sr_hbm.at[ir], sr_v.at[dst], sem))
        copies.append(pltpu.async_copy(nr_hbm.at[ir], nr_v.at[dst], sem))
    for c in copies:
        c.wait()

    zero = jnp.zeros((_D,), jnp.float32)

    def blk(i, carry):
        ap, aq, ar = carry
        ds = pl.ds(i * _D, _D)
        outv[ds] = spq_v[ds] + sr_v[ds]
        return (ap + np_v[ds], aq + nq_v[ds], ar + nr_v[ds])

    ap, aq, ar = lax.fori_loop(0, _BPW // _D, blk, (zero, zero, zero))
    partv[0] = ap
    partv[1] = aq
    partv[2] = ar

    pltpu.sync_copy(outv, out_hbm.at[pl.ds(base, _BPW)])
    pltpu.sync_copy(partv, part_hbm.at[wid])


def kernel(ps, qs, rs, P_table, Q_table, R_table, wp, wq, wr, fc_w):
    del qs
    fc = fc_w[0].astype(jnp.float32)
    fc_c = fc / jnp.maximum(jnp.sqrt(jnp.sum(fc * fc)), 1.0)

    def _cw(w):
        s = w[0, 0]
        return s / jnp.maximum(jnp.abs(s), 1.0)

    fcs = jnp.concatenate([
        jnp.stack([fc_c * _cw(wp), fc_c * _cw(wq), fc_c * _cw(wr),
                   jnp.ones((_D,), jnp.float32)]),
        jnp.zeros((4, _D), jnp.float32),
    ])

    spq, sr, np_, nq_, nr = pl.pallas_call(
        _tc_body,
        grid=(_GA,),
        in_specs=[
            pl.BlockSpec((_D, _VC), lambda c: (0, c)),
            pl.BlockSpec((_D, _VC), lambda c: (0, c)),
            pl.BlockSpec((_D, _VC), lambda c: (0, c)),
            pl.BlockSpec((8, _D), lambda c: (0, 0)),
        ],
        out_specs=[pl.BlockSpec((_VC,), lambda c: (c,))] * 5,
        out_shape=[jax.ShapeDtypeStruct((_V,), jnp.float32)] * 5,
    )(P_table.T, Q_table.T, R_table.T, fcs)

    ps2 = ps.astype(jnp.int32).reshape(_NW * _NCH, _CHUNK)
    rs2 = rs.astype(jnp.int32).reshape(_NW * _NCH, _CHUNK)

    out, parts = _sc_body(ps2, rs2, spq, sr, np_, nq_, nr)

    inferences = out.reshape(_B, 1)
    sums = parts.sum(axis=(0, 2))
    regs = _REG * (jnp.sqrt(sums[0]) + jnp.sqrt(sums[1]) + jnp.sqrt(sums[2]))
    return (inferences, regs)


# speedup vs baseline: 11.6889x; 1.0015x over previous
import functools

import jax
import jax.numpy as jnp
from jax import lax
from jax.experimental import pallas as pl
from jax.experimental.pallas import tpu as pltpu
from jax.experimental.pallas import tpu_sc as plsc

_V = 1000000
_B = 16384
_D = 16
_NC = 2
_NS = 16
_NW = _NC * _NS
_BPW = _B // _NW
_CHUNK = 128
_NCH = _BPW // _CHUNK
_VC = 131072
_GA = -(-_V // _VC)
_REG = 0.0001


def _tc_body(pt, qt, rt, w, spq, sr, np_, nq_, nr):
    p = pt[...]
    q = qt[...]
    r = rt[...]
    ww = w[...]
    dn = (((1,), (0,)), ((), ()))
    f32 = jnp.float32
    mp = lax.dot_general(ww, p, dn, preferred_element_type=f32)
    mq = lax.dot_general(ww, q, dn, preferred_element_type=f32)
    mr = lax.dot_general(ww, r, dn, preferred_element_type=f32)
    m2p = lax.dot_general(ww, p * p, dn, preferred_element_type=f32)
    m2q = lax.dot_general(ww, q * q, dn, preferred_element_type=f32)
    m2r = lax.dot_general(ww, r * r, dn, preferred_element_type=f32)
    spq[...] = mp[0] + mq[1]
    sr[...] = mr[2]
    np_[...] = m2p[3]
    nq_[...] = m2q[3]
    nr[...] = m2r[3]


@functools.partial(
    pl.kernel,
    out_type=[
        jax.ShapeDtypeStruct((_B,), jnp.float32),
        jax.ShapeDtypeStruct((_NW, 3, _D), jnp.float32),
    ],
    mesh=plsc.VectorSubcoreMesh(core_axis_name="c", subcore_axis_name="s"),
    scratch_types=[
        pltpu.VMEM((_NCH, _CHUNK), jnp.int32),
        pltpu.VMEM((_NCH, _CHUNK), jnp.int32),
        pltpu.VMEM((_BPW,), jnp.float32),
        pltpu.VMEM((_BPW,), jnp.float32),
        pltpu.VMEM((_BPW,), jnp.float32),
        pltpu.VMEM((_BPW,), jnp.float32),
        pltpu.VMEM((_BPW,), jnp.float32),
        pltpu.VMEM((_BPW,), jnp.float32),
        pltpu.VMEM((3, _D), jnp.float32),
        pltpu.SemaphoreType.DMA,
    ],
)
def _sc_body(ps_hbm, rs_hbm, spq_hbm, sr_hbm, np_hbm, nq_hbm, nr_hbm,
             out_hbm, part_hbm,
             idxp_v, idxr_v, spq_v, sr_v, np_v, nq_v, nr_v, outv, partv, sem):
    wid = lax.axis_index("s") * _NC + lax.axis_index("c")
    base = wid * _BPW

    pltpu.sync_copy(ps_hbm.at[pl.ds(wid * _NCH, _NCH)], idxp_v)
    pltpu.sync_copy(rs_hbm.at[pl.ds(wid * _NCH, _NCH)], idxr_v)

    copies = []
    for j in range(_NCH):
        dst = pl.ds(j * _CHUNK, _CHUNK)
        ip = idxp_v.at[j]
        ir = idxr_v.at[j]
        copies.append(pltpu.async_copy(spq_hbm.at[ip], spq_v.at[dst], sem))
        copies.append(pltpu.async_copy(np_hbm.at[ip], np_v.at[dst], sem))
        copies.append(pltpu.async_copy(nq_hbm.at[ip], nq_v.at[dst], sem))
        copies.append(pltpu.async_copy(sr_hbm.at[ir], sr_v.at[dst], sem))
        copies.append(pltpu.async_copy(nr_hbm.at[ir], nr_v.at[dst], sem))
    for c in copies:
        c.wait()

    zero = jnp.zeros((_D,), jnp.float32)

    def blk(i, carry):
        ap, aq, ar = carry
        ds = pl.ds(i * _D, _D)
        outv[ds] = spq_v[ds] + sr_v[ds]
        return (ap + np_v[ds], aq + nq_v[ds], ar + nr_v[ds])

    ap, aq, ar = lax.fori_loop(0, _BPW // _D, blk, (zero, zero, zero))
    partv[0] = ap
    partv[1] = aq
    partv[2] = ar

    pltpu.sync_copy(outv, out_hbm.at[pl.ds(base, _BPW)])
    pltpu.sync_copy(partv, part_hbm.at[wid])


def kernel(ps, qs, rs, P_table, Q_table, R_table, wp, wq, wr, fc_w):
    del qs
    fc = fc_w[0].astype(jnp.float32)
    fc_c = fc / jnp.maximum(jnp.sqrt(jnp.sum(fc * fc)), 1.0)

    def _cw(w):
        s = w[0, 0]
        return s / jnp.maximum(jnp.abs(s), 1.0)

    fcs = jnp.concatenate([
        jnp.stack([fc_c * _cw(wp), fc_c * _cw(wq), fc_c * _cw(wr),
                   jnp.ones((_D,), jnp.float32)]),
        jnp.zeros((4, _D), jnp.float32),
    ])

    spq, sr, np_, nq_, nr = pl.pallas_call(
        _tc_body,
        grid=(_GA,),
        in_specs=[
            pl.BlockSpec((_D, _VC), lambda c: (0, c)),
            pl.BlockSpec((_D, _VC), lambda c: (0, c)),
            pl.BlockSpec((_D, _VC), lambda c: (0, c)),
            pl.BlockSpec((8, _D), lambda c: (0, 0)),
        ],
        out_specs=[pl.BlockSpec((_VC,), lambda c: (c,))] * 5,
        out_shape=[jax.ShapeDtypeStruct((_V,), jnp.float32)] * 5,
    )(P_table.T, Q_table.T, R_table.T, fcs)

    ps2 = ps.astype(jnp.int32).reshape(_NW * _NCH, _CHUNK)
    rs2 = rs.astype(jnp.int32).reshape(_NW * _NCH, _CHUNK)

    out, parts = _sc_body(ps2, rs2, spq, sr, np_, nq_, nr)

    inferences = out.reshape(_B, 1)
    sums = parts.sum(axis=(0, 2))
    regs = _REG * (jnp.sqrt(sums[0]) + jnp.sqrt(sums[1]) + jnp.sqrt(sums[2]))
    return (inferences, regs)
